# baseline (speedup 1.0000x reference)
import functools

import jax
import jax.numpy as jnp
from jax import lax
from jax.experimental import pallas as pl
from jax.experimental.pallas import tpu as pltpu
from jax.experimental.pallas import tpu_sc as plsc

B, G, A, C = 16, 20, 5000, 20
AP = 5120
SB, LN = 8, 640
NCH = LN // 16
THRESHOLD = 0.5
BG = 20
IMG = 224.0
ALPHA = 0.25


GB = 8
NSTEP = B // GB


def _prep_body(targets_ref, anchors_ref, colp_ref, rm_ref, ra_ref):
    ax0 = anchors_ref[0]
    ay0 = anchors_ref[1]
    ax1 = anchors_ref[2]
    ay1 = anchors_ref[3]
    a_area = (ax1 - ax0) * (ay1 - ay0)

    aidx = (lax.broadcasted_iota(jnp.int32, (SB, LN), 0) * LN
            + lax.broadcasted_iota(jnp.int32, (SB, LN), 1)).astype(jnp.float32)

    for i in range(GB):
        rowmax = None
        rowarg = None
        for g in range(G):
            bx0 = targets_ref[i, 0, g]
            by0 = targets_ref[i, 1, g]
            bx1 = targets_ref[i, 2, g]
            by1 = targets_ref[i, 3, g]
            b_area = (bx1 - bx0) * (by1 - by0)
            ow = jnp.maximum(jnp.minimum(ax1, bx1) - jnp.maximum(bx0, ax0), 0.0)
            oh = jnp.maximum(jnp.minimum(ay1, by1) - jnp.maximum(by0, ay0), 0.0)
            overlaps = ow * oh
            union = (b_area + a_area) - overlaps
            iou = overlaps / union
            m8 = jnp.max(iou, axis=0, keepdims=True)
            i8 = jnp.min(jnp.where(iou == m8, aidx, 1e9), axis=0, keepdims=True)
            colp_ref[i, 0, g] = m8[0]
            colp_ref[i, 1, g] = i8[0]
            if g == 0:
                rowmax = iou
                rowarg = jnp.zeros_like(iou)
            else:
                upd = iou > rowmax
                rowmax = jnp.maximum(rowmax, iou)
                rowarg = jnp.where(upd, float(g), rowarg)

        rm_ref[i] = rowmax
        ra_ref[i] = rowarg


def _lane_rot(x, k):
    perm = jnp.bitwise_and(lax.iota(jnp.int32, 16) + (8 >> k), 15).reshape(16, 1)
    dnums = lax.GatherDimensionNumbers(
        offset_dims=(), collapsed_slice_dims=(0,), start_index_map=(0,))
    return lax.gather(x, perm, dnums, (1,),
                      mode=lax.GatherScatterMode.PROMISE_IN_BOUNDS)


def _lane_max_splat(x):
    for k in range(4):
        x = jnp.maximum(x, _lane_rot(x, k))
    return x


def _lane_min_splat(x):
    for k in range(4):
        x = jnp.minimum(x, _lane_rot(x, k))
    return x


@functools.partial(
    pl.kernel,
    out_type=[
        jax.ShapeDtypeStruct((B, G, 16), jnp.float32),
    ],
    mesh=plsc.VectorSubcoreMesh(core_axis_name="c", subcore_axis_name="s"),
    scratch_types=[
        pltpu.VMEM((2, G, LN), jnp.float32),
        pltpu.VMEM((G, 16), jnp.float32),
    ],
)
def _sc_match(colp_ref, bidx_o, cps, bidxS):
    c = lax.axis_index("c")
    s = lax.axis_index("s")

    @pl.when(s < 8)
    def _():
        b = c * 8 + s
        pltpu.sync_copy(colp_ref.at[b], cps)

        for g in range(G):
            def body(i, carry, g=g):
                cm, ci = carry
                off = i * 16
                m = cps[0, g, pl.ds(off, 16)]
                idx = cps[1, g, pl.ds(off, 16)]
                upd = m > cm
                return (jnp.maximum(cm, m), jnp.where(upd, idx, ci))

            cm0 = jnp.broadcast_to(jnp.float32(-1.0), (16,))
            ci0 = jnp.broadcast_to(jnp.float32(1e9), (16,))
            cm, ci = lax.fori_loop(0, NCH, body, (cm0, ci0), unroll=False)
            mx = _lane_max_splat(cm)
            bidxS[g] = _lane_min_splat(jnp.where(cm == mx, ci, 1e9))

        pltpu.sync_copy(bidxS, bidx_o.at[b])


def _loss_body(targets_ref, anchors_ref, pb_ref, pl_ref, rm_ref, ra_ref,
               bidx_ref, bb_ref, ll_ref):
    b = pl.program_id(0)

    aidx_i = (lax.broadcasted_iota(jnp.int32, (SB, LN), 0) * LN
              + lax.broadcasted_iota(jnp.int32, (SB, LN), 1))
    aidx = aidx_i.astype(jnp.float32)
    valid_f = (aidx_i < A).astype(jnp.float32)

    ax0 = anchors_ref[0]
    ay0 = anchors_ref[1]
    ax1 = anchors_ref[2]
    ay1 = anchors_ref[3]
    axn0, ayn0, axn1, ayn1 = ax0 / IMG, ay0 / IMG, ax1 / IMG, ay1 / IMG
    aw = axn1 - axn0
    ah = ayn1 - ayn0
    acx = axn0 + 0.5 * aw
    acy = ayn0 + 0.5 * ah

    bb_tot = 0.0
    ll_tot = 0.0
    for i in range(GB):
        is_best = aidx == bidx_ref[i, 0, 0]
        for g in range(1, G):
            is_best = is_best | (aidx == bidx_ref[i, g, 0])
        sel = (rm_ref[i] > THRESHOLD) | is_best
        sel_f = sel.astype(jnp.float32)
        n_sel = jnp.sum(sel_f)

        rowarg = ra_ref[i]
        cls = jnp.zeros((SB, LN), jnp.float32)
        tgt = [jnp.zeros((SB, LN), jnp.float32) for _ in range(4)]
        for g in range(G):
            match = (rowarg == float(g)).astype(jnp.float32)
            cls = cls + match * targets_ref[i, 4, g]
            for cc in range(4):
                tgt[cc] = tgt[cc] + match * (targets_ref[i, cc, g] / IMG)
        cls = jnp.where(sel, cls, float(BG))

        cx = acx + pb_ref[i, 0] * aw
        cy = acy + pb_ref[i, 1] * ah
        w = aw * jnp.exp(pb_ref[i, 2])
        h = ah * jnp.exp(pb_ref[i, 3])
        pred = [cx - 0.5 * w, cy - 0.5 * h, cx + 0.5 * w, cy + 0.5 * h]
        bb_sum = jnp.zeros((SB, LN), jnp.float32)
        for cc in range(4):
            d = pred[cc] - tgt[cc]
            ad = jnp.abs(d)
            bb_sum = bb_sum + jnp.where(ad < 1.0, 0.5 * d * d, ad - 0.5) * sel_f
        bb_tot = bb_tot + jnp.sum(bb_sum) / (n_sel * 4.0)

        ll_acc = jnp.zeros((SB, LN), jnp.float32)
        for cc in range(C):
            x = pl_ref[i, cc]
            pos = cls == float(cc)
            t = jnp.exp(-jnp.abs(x))
            l1p = jnp.log1p(t)
            inv = 1.0 / (1.0 + t)
            p = jnp.where(x >= 0.0, inv, t * inv)
            bce_neg = jnp.maximum(x, 0.0) + l1p
            term_neg = ((1.0 - ALPHA) * p) * bce_neg
            term_pos = (ALPHA * (1.0 - p)) * (bce_neg - x)
            ll_acc = ll_acc + jnp.where(pos, term_pos, term_neg)
        ll_tot = ll_tot + jnp.sum(ll_acc * valid_f) / float(A * C)

    @pl.when(b == 0)
    def _():
        bb_ref[...] = jnp.zeros((1, 1), jnp.float32)
        ll_ref[...] = jnp.zeros((1, 1), jnp.float32)

    bb_ref[...] = bb_ref[...] + bb_tot
    ll_ref[...] = ll_ref[...] + ll_tot


@jax.jit
def _ssd_loss(targets, anchors_t, pb_t, pl_t):
    colp, rm, ra = pl.pallas_call(
        _prep_body,
        grid=(NSTEP,),
        in_specs=[
            pl.BlockSpec((GB, 8, G), lambda b: (b, 0, 0)),
            pl.BlockSpec((4, SB, LN), lambda b: (0, 0, 0)),
        ],
        out_specs=[
            pl.BlockSpec((GB, 2, G, LN), lambda b: (b, 0, 0, 0)),
            pl.BlockSpec((GB, SB, LN), lambda b: (b, 0, 0)),
            pl.BlockSpec((GB, SB, LN), lambda b: (b, 0, 0)),
        ],
        out_shape=[
            jax.ShapeDtypeStruct((B, 2, G, LN), jnp.float32),
            jax.ShapeDtypeStruct((B, SB, LN), jnp.float32),
            jax.ShapeDtypeStruct((B, SB, LN), jnp.float32),
        ],
        compiler_params=pltpu.CompilerParams(
            dimension_semantics=("parallel",),
        ),
    )(targets, anchors_t)

    bidx, = _sc_match(colp)

    out = pl.pallas_call(
        _loss_body,
        grid=(NSTEP,),
        in_specs=[
            pl.BlockSpec((GB, 8, G), lambda b: (b, 0, 0)),
            pl.BlockSpec((4, SB, LN), lambda b: (0, 0, 0)),
            pl.BlockSpec((GB, 4, SB, LN), lambda b: (b, 0, 0, 0)),
            pl.BlockSpec((GB, C, SB, LN), lambda b: (b, 0, 0, 0)),
            pl.BlockSpec((GB, SB, LN), lambda b: (b, 0, 0)),
            pl.BlockSpec((GB, SB, LN), lambda b: (b, 0, 0)),
            pl.BlockSpec((GB, G, 16), lambda b: (b, 0, 0)),
        ],
        out_specs=[
            pl.BlockSpec((1, 1), lambda b: (0, 0)),
            pl.BlockSpec((1, 1), lambda b: (0, 0)),
        ],
        out_shape=[
            jax.ShapeDtypeStruct((1, 1), jnp.float32),
            jax.ShapeDtypeStruct((1, 1), jnp.float32),
        ],
        compiler_params=pltpu.CompilerParams(
            dimension_semantics=("arbitrary",),
        ),
    )(targets, anchors_t, pb_t, pl_t, rm, ra, bidx)
    return out[0][0, 0], out[1][0, 0]


def kernel(target_bb_batch, target_label_batch, pred_bb_batch, pred_label_batch, anchors):
    targets = jnp.concatenate(
        [jnp.transpose(target_bb_batch, (0, 2, 1)),
         target_label_batch.astype(jnp.float32)[:, None, :],
         jnp.zeros((B, 3, G), jnp.float32)], axis=1)
    anchors_t = jnp.pad(jnp.transpose(anchors, (1, 0)),
                        ((0, 0), (0, AP - A))).reshape(4, SB, LN)
    pb_t = jnp.pad(jnp.einsum('bac,dc->bda', pred_bb_batch,
                              jnp.eye(4, dtype=jnp.float32)),
                   ((0, 0), (0, 0), (0, AP - A))).reshape(B, 4, SB, LN)
    pl_t = jnp.pad(jnp.transpose(pred_label_batch, (0, 2, 1)),
                   ((0, 0), (0, 0), (0, AP - A))).reshape(B, C + 1, SB, LN)
    return _ssd_loss(targets, anchors_t, pb_t, pl_t)


# speedup vs baseline: 1.0118x; 1.0118x over previous
import functools

import jax
import jax.numpy as jnp
from jax import lax
from jax.experimental import pallas as pl
from jax.experimental.pallas import tpu as pltpu
from jax.experimental.pallas import tpu_sc as plsc

B, G, A, C = 16, 20, 5000, 20
AP = 5120
SB, LN = 8, 640
NCH = LN // 16
THRESHOLD = 0.5
BG = 20
IMG = 224.0
ALPHA = 0.25


GB = 4
NSTEP = B // GB


def _prep_body(targets_ref, anchors_ref, colp_ref, rm_ref, ra_ref):
    ax0 = anchors_ref[0]
    ay0 = anchors_ref[1]
    ax1 = anchors_ref[2]
    ay1 = anchors_ref[3]
    a_area = (ax1 - ax0) * (ay1 - ay0)

    aidx = (lax.broadcasted_iota(jnp.int32, (SB, LN), 0) * LN
            + lax.broadcasted_iota(jnp.int32, (SB, LN), 1)).astype(jnp.float32)

    for i in range(GB):
        rowmax = None
        rowarg = None
        for g in range(G):
            bx0 = targets_ref[i, 0, g]
            by0 = targets_ref[i, 1, g]
            bx1 = targets_ref[i, 2, g]
            by1 = targets_ref[i, 3, g]
            b_area = (bx1 - bx0) * (by1 - by0)
            ow = jnp.maximum(jnp.minimum(ax1, bx1) - jnp.maximum(bx0, ax0), 0.0)
            oh = jnp.maximum(jnp.minimum(ay1, by1) - jnp.maximum(by0, ay0), 0.0)
            overlaps = ow * oh
            union = (b_area + a_area) - overlaps
            iou = overlaps / union
            m8 = jnp.max(iou, axis=0, keepdims=True)
            i8 = jnp.min(jnp.where(iou == m8, aidx, 1e9), axis=0, keepdims=True)
            colp_ref[i, 0, g] = m8[0]
            colp_ref[i, 1, g] = i8[0]
            if g == 0:
                rowmax = iou
                rowarg = jnp.zeros_like(iou)
            else:
                upd = iou > rowmax
                rowmax = jnp.maximum(rowmax, iou)
                rowarg = jnp.where(upd, float(g), rowarg)

        rm_ref[i] = rowmax
        ra_ref[i] = rowarg


def _lane_rot(x, k):
    perm = jnp.bitwise_and(lax.iota(jnp.int32, 16) + (8 >> k), 15).reshape(16, 1)
    dnums = lax.GatherDimensionNumbers(
        offset_dims=(), collapsed_slice_dims=(0,), start_index_map=(0,))
    return lax.gather(x, perm, dnums, (1,),
                      mode=lax.GatherScatterMode.PROMISE_IN_BOUNDS)


def _lane_max_splat(x):
    for k in range(4):
        x = jnp.maximum(x, _lane_rot(x, k))
    return x


def _lane_min_splat(x):
    for k in range(4):
        x = jnp.minimum(x, _lane_rot(x, k))
    return x


@functools.partial(
    pl.kernel,
    out_type=[
        jax.ShapeDtypeStruct((B, G, 16), jnp.float32),
    ],
    mesh=plsc.VectorSubcoreMesh(core_axis_name="c", subcore_axis_name="s"),
    scratch_types=[
        pltpu.VMEM((2, G, LN), jnp.float32),
        pltpu.VMEM((G, 16), jnp.float32),
    ],
)
def _sc_match(colp_ref, bidx_o, cps, bidxS):
    c = lax.axis_index("c")
    s = lax.axis_index("s")

    @pl.when(s < 8)
    def _():
        b = c * 8 + s
        pltpu.sync_copy(colp_ref.at[b], cps)

        for g in range(G):
            def body(i, carry, g=g):
                cm, ci = carry
                off = i * 16
                m = cps[0, g, pl.ds(off, 16)]
                idx = cps[1, g, pl.ds(off, 16)]
                upd = m > cm
                return (jnp.maximum(cm, m), jnp.where(upd, idx, ci))

            cm0 = jnp.broadcast_to(jnp.float32(-1.0), (16,))
            ci0 = jnp.broadcast_to(jnp.float32(1e9), (16,))
            cm, ci = lax.fori_loop(0, NCH, body, (cm0, ci0), unroll=False)
            mx = _lane_max_splat(cm)
            bidxS[g] = _lane_min_splat(jnp.where(cm == mx, ci, 1e9))

        pltpu.sync_copy(bidxS, bidx_o.at[b])


def _loss_body(targets_ref, anchors_ref, pb_ref, pl_ref, rm_ref, ra_ref,
               bidx_ref, bb_ref, ll_ref):
    b = pl.program_id(0)

    aidx_i = (lax.broadcasted_iota(jnp.int32, (SB, LN), 0) * LN
              + lax.broadcasted_iota(jnp.int32, (SB, LN), 1))
    aidx = aidx_i.astype(jnp.float32)
    valid_f = (aidx_i < A).astype(jnp.float32)

    ax0 = anchors_ref[0]
    ay0 = anchors_ref[1]
    ax1 = anchors_ref[2]
    ay1 = anchors_ref[3]
    axn0, ayn0, axn1, ayn1 = ax0 / IMG, ay0 / IMG, ax1 / IMG, ay1 / IMG
    aw = axn1 - axn0
    ah = ayn1 - ayn0
    acx = axn0 + 0.5 * aw
    acy = ayn0 + 0.5 * ah

    bb_tot = 0.0
    ll_tot = 0.0
    for i in range(GB):
        is_best = aidx == bidx_ref[i, 0, 0]
        for g in range(1, G):
            is_best = is_best | (aidx == bidx_ref[i, g, 0])
        sel = (rm_ref[i] > THRESHOLD) | is_best
        sel_f = sel.astype(jnp.float32)
        n_sel = jnp.sum(sel_f)

        rowarg = ra_ref[i]
        cls = jnp.zeros((SB, LN), jnp.float32)
        tgt = [jnp.zeros((SB, LN), jnp.float32) for _ in range(4)]
        for g in range(G):
            match = (rowarg == float(g)).astype(jnp.float32)
            cls = cls + match * targets_ref[i, 4, g]
            for cc in range(4):
                tgt[cc] = tgt[cc] + match * (targets_ref[i, cc, g] / IMG)
        cls = jnp.where(sel, cls, float(BG))

        cx = acx + pb_ref[i, 0] * aw
        cy = acy + pb_ref[i, 1] * ah
        w = aw * jnp.exp(pb_ref[i, 2])
        h = ah * jnp.exp(pb_ref[i, 3])
        pred = [cx - 0.5 * w, cy - 0.5 * h, cx + 0.5 * w, cy + 0.5 * h]
        bb_sum = jnp.zeros((SB, LN), jnp.float32)
        for cc in range(4):
            d = pred[cc] - tgt[cc]
            ad = jnp.abs(d)
            bb_sum = bb_sum + jnp.where(ad < 1.0, 0.5 * d * d, ad - 0.5) * sel_f
        bb_tot = bb_tot + jnp.sum(bb_sum) / (n_sel * 4.0)

        ll_acc = jnp.zeros((SB, LN), jnp.float32)
        for cc in range(C):
            x = pl_ref[i, cc]
            pos = cls == float(cc)
            t = jnp.exp(-jnp.abs(x))
            l1p = jnp.log1p(t)
            inv = 1.0 / (1.0 + t)
            p = jnp.where(x >= 0.0, inv, t * inv)
            bce_neg = jnp.maximum(x, 0.0) + l1p
            term_neg = ((1.0 - ALPHA) * p) * bce_neg
            term_pos = (ALPHA * (1.0 - p)) * (bce_neg - x)
            ll_acc = ll_acc + jnp.where(pos, term_pos, term_neg)
        ll_tot = ll_tot + jnp.sum(ll_acc * valid_f) / float(A * C)

    @pl.when(b == 0)
    def _():
        bb_ref[...] = jnp.zeros((1, 1), jnp.float32)
        ll_ref[...] = jnp.zeros((1, 1), jnp.float32)

    bb_ref[...] = bb_ref[...] + bb_tot
    ll_ref[...] = ll_ref[...] + ll_tot


@jax.jit
def _ssd_loss(targets, anchors_t, pb_t, pl_t):
    colp, rm, ra = pl.pallas_call(
        _prep_body,
        grid=(NSTEP,),
        in_specs=[
            pl.BlockSpec((GB, 8, G), lambda b: (b, 0, 0)),
            pl.BlockSpec((4, SB, LN), lambda b: (0, 0, 0)),
        ],
        out_specs=[
            pl.BlockSpec((GB, 2, G, LN), lambda b: (b, 0, 0, 0)),
            pl.BlockSpec((GB, SB, LN), lambda b: (b, 0, 0)),
            pl.BlockSpec((GB, SB, LN), lambda b: (b, 0, 0)),
        ],
        out_shape=[
            jax.ShapeDtypeStruct((B, 2, G, LN), jnp.float32),
            jax.ShapeDtypeStruct((B, SB, LN), jnp.float32),
            jax.ShapeDtypeStruct((B, SB, LN), jnp.float32),
        ],
        compiler_params=pltpu.CompilerParams(
            dimension_semantics=("parallel",),
        ),
    )(targets, anchors_t)

    bidx, = _sc_match(colp)

    out = pl.pallas_call(
        _loss_body,
        grid=(NSTEP,),
        in_specs=[
            pl.BlockSpec((GB, 8, G), lambda b: (b, 0, 0)),
            pl.BlockSpec((4, SB, LN), lambda b: (0, 0, 0)),
            pl.BlockSpec((GB, 4, SB, LN), lambda b: (b, 0, 0, 0)),
            pl.BlockSpec((GB, C, SB, LN), lambda b: (b, 0, 0, 0)),
            pl.BlockSpec((GB, SB, LN), lambda b: (b, 0, 0)),
            pl.BlockSpec((GB, SB, LN), lambda b: (b, 0, 0)),
            pl.BlockSpec((GB, G, 16), lambda b: (b, 0, 0)),
        ],
        out_specs=[
            pl.BlockSpec((1, 1), lambda b: (0, 0)),
            pl.BlockSpec((1, 1), lambda b: (0, 0)),
        ],
        out_shape=[
            jax.ShapeDtypeStruct((1, 1), jnp.float32),
            jax.ShapeDtypeStruct((1, 1), jnp.float32),
        ],
        compiler_params=pltpu.CompilerParams(
            dimension_semantics=("arbitrary",),
        ),
    )(targets, anchors_t, pb_t, pl_t, rm, ra, bidx)
    return out[0][0, 0], out[1][0, 0]


def kernel(target_bb_batch, target_label_batch, pred_bb_batch, pred_label_batch, anchors):
    targets = jnp.concatenate(
        [jnp.transpose(target_bb_batch, (0, 2, 1)),
         target_label_batch.astype(jnp.float32)[:, None, :],
         jnp.zeros((B, 3, G), jnp.float32)], axis=1)
    anchors_t = jnp.pad(jnp.transpose(anchors, (1, 0)),
                        ((0, 0), (0, AP - A))).reshape(4, SB, LN)
    pb_t = jnp.pad(jnp.einsum('bac,dc->bda', pred_bb_batch,
                              jnp.eye(4, dtype=jnp.float32)),
                   ((0, 0), (0, 0), (0, AP - A))).reshape(B, 4, SB, LN)
    pl_t = jnp.pad(jnp.transpose(pred_label_batch, (0, 2, 1)),
                   ((0, 0), (0, 0), (0, AP - A))).reshape(B, C + 1, SB, LN)
    return _ssd_loss(targets, anchors_t, pb_t, pl_t)
